# Initial kernel scaffold; baseline (speedup 1.0000x reference)
import jax
import jax.numpy as jnp
from jax.experimental import pallas as pl


def kernel(atomic_num, chirality, degree, formal_charge, num_h, hybridization, table_atomic_num, table_chirality, table_degree, table_formal_charge, table_num_h, table_hybridization):
    raise NotImplementedError("write your pallas kernel here")


# speedup vs baseline: 24.6868x; 24.6868x over previous
import functools

import jax
import jax.numpy as jnp
from jax import lax
from jax.experimental import pallas as pl
from jax.experimental.pallas import tpu as pltpu
from jax.experimental.pallas import tpu_sc as plsc

_SIZES = (119, 5, 12, 12, 10, 8)
_N = 100000
_NC, _NS = 2, 16
_NW = _NC * _NS
_CHUNK = 3136
_NPAD = _NW * _CHUNK
_VPW = _CHUNK // 16


def _rowsum_body(t0, t1, t2, t3, t4, t5, out_ref):
    sums = [jnp.sum(t[...], axis=1) for t in (t0, t1, t2, t3, t4, t5)]
    sums += [jnp.zeros((128,), jnp.float32)] * 2
    out_ref[...] = jnp.stack(sums)


_rowsum_call = pl.pallas_call(
    _rowsum_body,
    out_shape=jax.ShapeDtypeStruct((8, 128), jnp.float32),
)


def _sc_body(g_hbm, f0, f1, f2, f3, f4, f5, out_hbm, g_v, chunk_v, acc_v):
    wid = lax.axis_index("s") * _NC + lax.axis_index("c")
    pltpu.sync_copy(g_hbm, g_v)
    acc = jnp.zeros((16,), jnp.float32)
    for fi, fref in enumerate((f0, f1, f2, f3, f4, f5)):
        pltpu.sync_copy(fref.at[wid], chunk_v)

        def body(i, a, fi=fi):
            x = chunk_v[i]
            idx = x.astype(jnp.int32) + (128 * fi)
            return a + x * plsc.load_gather(g_v, [idx])

        acc = lax.fori_loop(0, _VPW, body, acc, unroll=4)
    acc_v[0] = acc
    pltpu.sync_copy(acc_v, out_hbm.at[wid])


@functools.cache
def _sc_call():
    return functools.partial(
        pl.kernel,
        out_type=jax.ShapeDtypeStruct((_NW, 1, 16), jnp.float32),
        mesh=plsc.VectorSubcoreMesh(core_axis_name="c", subcore_axis_name="s",
                                    num_cores=_NC, num_subcores=_NS),
        compiler_params=pltpu.CompilerParams(needs_layout_passes=False),
        scratch_types=[
            pltpu.VMEM((1024,), jnp.float32),
            pltpu.VMEM((_VPW, 16), jnp.float32),
            pltpu.VMEM((1, 16), jnp.float32),
        ],
    )(_sc_body)


def kernel(atomic_num, chirality, degree, formal_charge, num_h, hybridization,
           table_atomic_num, table_chirality, table_degree, table_formal_charge,
           table_num_h, table_hybridization):
    feats = (atomic_num, chirality, degree, formal_charge, num_h, hybridization)
    tables = (table_atomic_num, table_chirality, table_degree,
              table_formal_charge, table_num_h, table_hybridization)
    tpad = [jnp.pad(t, ((0, 128 - s), (0, 0))) for t, s in zip(tables, _SIZES)]
    g = _rowsum_call(*tpad).reshape(1024)
    fpad = [jnp.pad(f, (0, _NPAD - _N)).reshape(_NW, _VPW, 16) for f in feats]
    partials = _sc_call()(g, *fpad)
    return jnp.sum(partials)


# speedup vs baseline: 39.4505x; 1.5980x over previous
import functools

import jax
import jax.numpy as jnp
from jax import lax
from jax.experimental import pallas as pl
from jax.experimental.pallas import tpu as pltpu
from jax.experimental.pallas import tpu_sc as plsc

_SIZES = (119, 5, 12, 12, 10, 8)
_D = 64
_N = 100000
_NC, _NS = 2, 16
_NW = _NC * _NS
_CHUNK = 3136
_TAIL = _N - (_NW - 1) * _CHUNK
_VPW = _CHUNK // 16
_VPT = _TAIL // 16
_TOFF = tuple(sum(s * _D for s in _SIZES[:i]) for i in range(6))
_TTOT = sum(s * _D for s in _SIZES)


def _sc_body(tcat_hbm, f0, f1, f2, f3, f4, f5, out_hbm, t_v, g_v, chunk_v, acc_v):
    wid = lax.axis_index("s") * _NC + lax.axis_index("c")
    pltpu.sync_copy(tcat_hbm, t_v)

    lane = lax.iota(jnp.int32, 16)
    for fi, size in enumerate(_SIZES):
        for j in range((size + 15) // 16):
            rows = j * 16 + lane
            base = _TOFF[fi] + rows * _D
            mask = rows < size

            def dbody(d, a, base=base, mask=mask):
                return a + plsc.load_gather(t_v, [base + d], mask=mask)

            acc = lax.fori_loop(0, _D, dbody, jnp.zeros((16,), jnp.float32),
                                unroll=4)
            g_v[pl.ds(128 * fi + 16 * j, 16)] = acc

    acc = jnp.zeros((16,), jnp.float32)
    for fi, fref in enumerate((f0, f1, f2, f3, f4, f5)):

        def body(i, a, fi=fi):
            x = chunk_v[pl.ds(i * 16, 16)]
            idx = x.astype(jnp.int32) + (128 * fi)
            return a + x * plsc.load_gather(g_v, [idx])

        @pl.when(wid < _NW - 1)
        def _():
            pltpu.sync_copy(fref.at[pl.ds(wid * _CHUNK, _CHUNK)],
                            chunk_v.at[pl.ds(0, _CHUNK)])

        @pl.when(wid == _NW - 1)
        def _():
            pltpu.sync_copy(fref.at[pl.ds(wid * _CHUNK, _TAIL)],
                            chunk_v.at[pl.ds(0, _TAIL)])

        acc = lax.fori_loop(0, _VPT, body, acc, unroll=4)

        @pl.when(wid < _NW - 1)
        def _():
            a2 = lax.fori_loop(_VPT, _VPW, body, jnp.zeros((16,), jnp.float32),
                               unroll=4)
            acc_v[...] = a2

        @pl.when(wid == _NW - 1)
        def _():
            acc_v[...] = jnp.zeros((16,), jnp.float32)

        acc = acc + acc_v[...]
    acc_v[...] = acc
    pltpu.sync_copy(acc_v, out_hbm.at[pl.ds(wid * 16, 16)])


@functools.cache
def _sc_call():
    return functools.partial(
        pl.kernel,
        out_type=jax.ShapeDtypeStruct((_NW * 16,), jnp.float32),
        mesh=plsc.VectorSubcoreMesh(core_axis_name="c", subcore_axis_name="s",
                                    num_cores=_NC, num_subcores=_NS),
        compiler_params=pltpu.CompilerParams(needs_layout_passes=False,
                                             use_tc_tiling_on_sc=False),
        scratch_types=[
            pltpu.VMEM((_TTOT,), jnp.float32),
            pltpu.VMEM((1024,), jnp.float32),
            pltpu.VMEM((_CHUNK,), jnp.float32),
            pltpu.VMEM((16,), jnp.float32),
        ],
    )(_sc_body)


def kernel(atomic_num, chirality, degree, formal_charge, num_h, hybridization,
           table_atomic_num, table_chirality, table_degree, table_formal_charge,
           table_num_h, table_hybridization):
    feats = (atomic_num, chirality, degree, formal_charge, num_h, hybridization)
    tables = (table_atomic_num, table_chirality, table_degree,
              table_formal_charge, table_num_h, table_hybridization)
    tcat = (jnp.concatenate([t.reshape(-1) for t in tables])
            .astype(jnp.bfloat16).astype(jnp.float32))
    partials = _sc_call()(tcat, *feats)
    return jnp.sum(partials)


# speedup vs baseline: 43.3123x; 1.0979x over previous
import functools

import jax
import jax.numpy as jnp
from jax import lax
from jax.experimental import pallas as pl
from jax.experimental.pallas import tpu as pltpu
from jax.experimental.pallas import tpu_sc as plsc

_SIZES = (119, 5, 12, 12, 10, 8)
_D = 64
_N = 100000
_NC, _NS = 2, 16
_NW = _NC * _NS
_CHUNK = 3136
_TAIL = _N - (_NW - 1) * _CHUNK
_VPW = _CHUNK // 16
_VPT = _TAIL // 16
_TOFF = tuple(sum(s * _D for s in _SIZES[:i]) for i in range(6))
_TTOT = sum(s * _D for s in _SIZES)


def _sc_body(tcat_hbm, f0, f1, f2, f3, f4, f5, out_hbm,
             t_v, g_v, c0, c1, c2, c3, c4, c5, acc_v, *sems):
    wid = lax.axis_index("s") * _NC + lax.axis_index("c")
    frefs = (f0, f1, f2, f3, f4, f5)
    chunks = (c0, c1, c2, c3, c4, c5)
    zeros = jnp.zeros((16,), jnp.float32)

    copies = []
    for fref, chunk, sem in zip(frefs, chunks, sems):
        cf = pltpu.make_async_copy(fref.at[pl.ds(wid * _CHUNK, _CHUNK)],
                                   chunk.at[pl.ds(0, _CHUNK)], sem)
        ct = pltpu.make_async_copy(fref.at[pl.ds(wid * _CHUNK, _TAIL)],
                                   chunk.at[pl.ds(0, _TAIL)], sem)
        copies.append((cf, ct))

        @pl.when(wid < _NW - 1)
        def _(cf=cf):
            cf.start()

        @pl.when(wid == _NW - 1)
        def _(ct=ct):
            ct.start()

    pltpu.sync_copy(tcat_hbm, t_v)

    lane = lax.iota(jnp.int32, 16)
    for fi, size in enumerate(_SIZES):
        for j in range((size + 15) // 16):
            base = _TOFF[fi] + (j * 16 + lane) * _D

            def dbody(d, accs, base=base):
                return tuple(
                    a + plsc.load_gather(t_v, [base + (4 * d + k)])
                    for k, a in enumerate(accs))

            accs = lax.fori_loop(0, _D // 4, dbody, (zeros,) * 4, unroll=2)
            g_v[pl.ds(128 * fi + 16 * j, 16)] = sum(accs)

    @pl.when(wid == _NW - 1)
    def _():
        for chunk in chunks:
            for j in range(_VPT, _VPW):
                chunk[pl.ds(j * 16, 16)] = zeros

    acc_f = zeros
    for fi, (chunk, (cf, ct)) in enumerate(zip(chunks, copies)):
        @pl.when(wid < _NW - 1)
        def _(cf=cf):
            cf.wait()

        @pl.when(wid == _NW - 1)
        def _(ct=ct):
            ct.wait()

        gseg = g_v.at[pl.ds(128 * fi, 128)]

        def body(i, accs, chunk=chunk, gseg=gseg):
            out = []
            for k, a in enumerate(accs):
                x = chunk[pl.ds((4 * i + k) * 16, 16)]
                out.append(a + x * plsc.load_gather(gseg, [x.astype(jnp.int32)]))
            return tuple(out)

        accs = lax.fori_loop(0, _VPW // 4, body, (zeros,) * 4, unroll=2)
        acc_f = acc_f + (accs[0] + accs[1]) + (accs[2] + accs[3])
    acc_v[...] = acc_f
    pltpu.sync_copy(acc_v, out_hbm.at[pl.ds(wid * 16, 16)])


@functools.cache
def _sc_call():
    return functools.partial(
        pl.kernel,
        out_type=jax.ShapeDtypeStruct((_NW * 16,), jnp.float32),
        mesh=plsc.VectorSubcoreMesh(core_axis_name="c", subcore_axis_name="s",
                                    num_cores=_NC, num_subcores=_NS),
        compiler_params=pltpu.CompilerParams(needs_layout_passes=False,
                                             use_tc_tiling_on_sc=False),
        scratch_types=[
            pltpu.VMEM((_TTOT,), jnp.float32),
            pltpu.VMEM((1024,), jnp.float32),
        ] + [pltpu.VMEM((_CHUNK,), jnp.float32)] * 6 + [
            pltpu.VMEM((16,), jnp.float32),
        ] + [pltpu.SemaphoreType.DMA] * 6,
    )(_sc_body)


def kernel(atomic_num, chirality, degree, formal_charge, num_h, hybridization,
           table_atomic_num, table_chirality, table_degree, table_formal_charge,
           table_num_h, table_hybridization):
    feats = (atomic_num, chirality, degree, formal_charge, num_h, hybridization)
    tables = (table_atomic_num, table_chirality, table_degree,
              table_formal_charge, table_num_h, table_hybridization)
    tcat = (jnp.concatenate([t.reshape(-1) for t in tables])
            .astype(jnp.bfloat16).astype(jnp.float32))
    partials = _sc_call()(tcat, *feats)
    return jnp.sum(partials)


# speedup vs baseline: 48.3230x; 1.1157x over previous
import functools

import jax
import jax.numpy as jnp
from jax import lax
from jax.experimental import pallas as pl
from jax.experimental.pallas import tpu as pltpu
from jax.experimental.pallas import tpu_sc as plsc

_SIZES = (119, 5, 12, 12, 10, 8)
_D = 64
_N = 100000
_NC, _NS = 2, 16
_NW = _NC * _NS
_CHUNK = 3136
_TAIL = _N - (_NW - 1) * _CHUNK
_VPW = _CHUNK // 16
_VPT = _TAIL // 16
_TOFF = tuple(sum(s * _D for s in _SIZES[:i]) for i in range(6))
_TTOT = sum(s * _D for s in _SIZES)


def _sc_body(tcat_hbm, f0, f1, f2, f3, f4, f5, out_hbm,
             t_v, g_v, c0, c1, c2, c3, c4, c5, acc_v, *sems):
    wid = lax.axis_index("s") * _NC + lax.axis_index("c")
    frefs = (f0, f1, f2, f3, f4, f5)
    chunks = (c0, c1, c2, c3, c4, c5)
    zeros = jnp.zeros((16,), jnp.float32)

    copies = []
    for fref, chunk, sem in zip(frefs, chunks, sems):
        cf = pltpu.make_async_copy(fref.at[pl.ds(wid * _CHUNK, _CHUNK)],
                                   chunk.at[pl.ds(0, _CHUNK)], sem)
        ct = pltpu.make_async_copy(fref.at[pl.ds(wid * _CHUNK, _TAIL)],
                                   chunk.at[pl.ds(0, _TAIL)], sem)
        copies.append((cf, ct))

        @pl.when(wid < _NW - 1)
        def _(cf=cf):
            cf.start()

        @pl.when(wid == _NW - 1)
        def _(ct=ct):
            ct.start()

    pltpu.sync_copy(tcat_hbm, t_v)

    lane = lax.iota(jnp.int32, 16)
    for fi, size in enumerate(_SIZES):
        for j in range((size + 15) // 16):
            base = _TOFF[fi] + j * 16 + lane

            def dbody(d, accs, base=base, size=size):
                return tuple(
                    a + plsc.load_gather(t_v, [base + (4 * d + k) * size])
                    for k, a in enumerate(accs))

            accs = lax.fori_loop(0, _D // 4, dbody, (zeros,) * 4, unroll=2)
            g_v[pl.ds(128 * fi + 16 * j, 16)] = sum(accs)

    @pl.when(wid == _NW - 1)
    def _():
        for chunk in chunks:
            for j in range(_VPT, _VPW):
                chunk[pl.ds(j * 16, 16)] = zeros

    acc_f = zeros
    for fi, (chunk, (cf, ct)) in enumerate(zip(chunks, copies)):
        @pl.when(wid < _NW - 1)
        def _(cf=cf):
            cf.wait()

        @pl.when(wid == _NW - 1)
        def _(ct=ct):
            ct.wait()

        gseg = g_v.at[pl.ds(128 * fi, 128)]

        def body(i, accs, chunk=chunk, gseg=gseg):
            out = []
            for k, a in enumerate(accs):
                x = chunk[pl.ds((4 * i + k) * 16, 16)]
                out.append(a + x * plsc.load_gather(gseg, [x.astype(jnp.int32)]))
            return tuple(out)

        accs = lax.fori_loop(0, _VPW // 4, body, (zeros,) * 4, unroll=2)
        acc_f = acc_f + (accs[0] + accs[1]) + (accs[2] + accs[3])
    acc_v[...] = acc_f
    pltpu.sync_copy(acc_v, out_hbm.at[pl.ds(wid * 16, 16)])


@functools.cache
def _sc_call():
    return functools.partial(
        pl.kernel,
        out_type=jax.ShapeDtypeStruct((_NW * 16,), jnp.float32),
        mesh=plsc.VectorSubcoreMesh(core_axis_name="c", subcore_axis_name="s",
                                    num_cores=_NC, num_subcores=_NS),
        compiler_params=pltpu.CompilerParams(needs_layout_passes=False,
                                             use_tc_tiling_on_sc=False),
        scratch_types=[
            pltpu.VMEM((_TTOT,), jnp.float32),
            pltpu.VMEM((1024,), jnp.float32),
        ] + [pltpu.VMEM((_CHUNK,), jnp.float32)] * 6 + [
            pltpu.VMEM((16,), jnp.float32),
        ] + [pltpu.SemaphoreType.DMA] * 6,
    )(_sc_body)


def kernel(atomic_num, chirality, degree, formal_charge, num_h, hybridization,
           table_atomic_num, table_chirality, table_degree, table_formal_charge,
           table_num_h, table_hybridization):
    feats = (atomic_num, chirality, degree, formal_charge, num_h, hybridization)
    tables = (table_atomic_num, table_chirality, table_degree,
              table_formal_charge, table_num_h, table_hybridization)
    tcat = (jnp.concatenate([t.T.reshape(-1) for t in tables])
            .astype(jnp.bfloat16).astype(jnp.float32))
    partials = _sc_call()(tcat, *feats)
    return jnp.sum(partials)
